# Initial kernel scaffold; baseline (speedup 1.0000x reference)
import jax
import jax.numpy as jnp
from jax.experimental import pallas as pl


def kernel(inputs, embedding):
    raise NotImplementedError("write your pallas kernel here")


# speedup vs baseline: 1.3080x; 1.3080x over previous
import functools

import jax
import jax.numpy as jnp
from jax import lax
from jax.experimental import pallas as pl
from jax.experimental.pallas import tpu as pltpu
from jax.experimental.pallas import tpu_sc as plsc

NC = 2
NS = 16
NW = NC * NS
CH = 128


def _gather_body(nch, idx_hbm, table_hbm, out_hbm, idx_v, rows_v, gsem):
    wid = lax.axis_index("s") * NC + lax.axis_index("c")
    base = wid * (nch * CH)
    pltpu.sync_copy(idx_hbm.at[wid], idx_v)

    def chunk(j, carry):
        pltpu.async_copy(table_hbm.at[idx_v.at[j]], rows_v, gsem).wait()
        pltpu.sync_copy(rows_v, out_hbm.at[pl.ds(base + j * CH, CH)])
        return carry

    lax.fori_loop(0, nch, chunk, 0)


def kernel(inputs, embedding):
    bt, s = inputs.shape
    v, d = embedding.shape
    b = bt * s
    nch = b // (NW * CH)
    assert b == NW * nch * CH

    idx = inputs.reshape(NW, nch, CH).astype(jnp.int32)
    mesh = plsc.VectorSubcoreMesh(core_axis_name="c", subcore_axis_name="s")
    k = pl.kernel(
        functools.partial(_gather_body, nch),
        out_type=jax.ShapeDtypeStruct((b, d), jnp.float32),
        mesh=mesh,
        scratch_types=[
            pltpu.VMEM((nch, CH), jnp.int32),
            pltpu.VMEM((CH, d), jnp.float32),
            pltpu.SemaphoreType.DMA,
        ],
        compiler_params=pltpu.CompilerParams(use_tc_tiling_on_sc=False),
    )
    out = k(idx, embedding)
    return out.reshape(bt, s, d)


# speedup vs baseline: 1.4693x; 1.1234x over previous
import functools

import jax
import jax.numpy as jnp
from jax import lax
from jax.experimental import pallas as pl
from jax.experimental.pallas import tpu as pltpu
from jax.experimental.pallas import tpu_sc as plsc

NC = 2
NS = 16
NW = NC * NS
CH = 128
K = 4


def _gather_body(nch, idx_hbm, table_hbm, out_hbm, idx_v, rows_v, gsem, ssem):
    wid = lax.axis_index("s") * NC + lax.axis_index("c")
    base = wid * (nch * CH)
    ngroups = nch // K
    pltpu.sync_copy(idx_hbm.at[wid], idx_v)

    def store_desc(g, b):
        p = lax.rem(g, 2)
        j = g * K + b
        return pltpu.make_async_copy(
            rows_v.at[p, b], out_hbm.at[pl.ds(base + j * CH, CH)], ssem)

    def gather_desc(g, b):
        p = lax.rem(g, 2)
        j = g * K + b
        return pltpu.make_async_copy(
            table_hbm.at[idx_v.at[j]], rows_v.at[p, b], gsem)

    def group(g, carry):
        @pl.when(g >= 2)
        def _():
            for b in range(K):
                store_desc(g - 2, b).wait()
        for b in range(K):
            gather_desc(g, b).start()
        for b in range(K):
            gather_desc(g, b).wait()
        for b in range(K):
            store_desc(g, b).start()
        return carry

    lax.fori_loop(0, ngroups, group, 0)
    for b in range(K):
        store_desc(ngroups - 2, b).wait()
    for b in range(K):
        store_desc(ngroups - 1, b).wait()


def kernel(inputs, embedding):
    bt, s = inputs.shape
    v, d = embedding.shape
    b = bt * s
    nch = b // (NW * CH)
    assert b == NW * nch * CH and nch % (2 * K) == 0

    idx = inputs.reshape(NW, nch, CH).astype(jnp.int32)
    mesh = plsc.VectorSubcoreMesh(core_axis_name="c", subcore_axis_name="s")
    k = pl.kernel(
        functools.partial(_gather_body, nch),
        out_type=jax.ShapeDtypeStruct((b, d), jnp.float32),
        mesh=mesh,
        scratch_types=[
            pltpu.VMEM((nch, CH), jnp.int32),
            pltpu.VMEM((2, K, CH, d), jnp.float32),
            pltpu.SemaphoreType.DMA,
            pltpu.SemaphoreType.DMA,
        ],
        compiler_params=pltpu.CompilerParams(use_tc_tiling_on_sc=False),
    )
    out = k(idx, embedding)
    return out.reshape(bt, s, d)
